# Initial kernel scaffold; baseline (speedup 1.0000x reference)
import jax
import jax.numpy as jnp
from jax.experimental import pallas as pl


def kernel(vertices, W1, b1, W2, b2, W3, b3):
    raise NotImplementedError("write your pallas kernel here")


# speedup vs baseline: 2.6614x; 2.6614x over previous
import functools
import jax
import jax.numpy as jnp
from jax.experimental import pallas as pl
from jax.experimental.pallas import tpu as pltpu

N_PTS = 50000
N_PAD = 50048
WCOLS = N_PAD // 8
N_BALLS = 1024
K_NBR = 64
RADIUS = jnp.float32(0.2)


def _fps_body(x_ref, y_ref, z_ref, l_ref, cent_ref):
    X = x_ref[...]
    Y = y_ref[...]
    Z = z_ref[...]
    iota = jax.lax.broadcasted_iota(jnp.int32, (8, WCOLS), 0) * WCOLS + \
        jax.lax.broadcasted_iota(jnp.int32, (8, WCOLS), 1)
    valid = iota < N_PTS
    lane32 = jax.lax.broadcasted_iota(jnp.int32, (1, 32), 1)

    def extract(n):
        row = l_ref[pl.ds(n // 8, 1), :]
        q4 = (n % 8) * 4
        cx = jnp.sum(jnp.where(lane32 == q4, row, 0.0))
        cy = jnp.sum(jnp.where(lane32 == q4 + 1, row, 0.0))
        cz = jnp.sum(jnp.where(lane32 == q4 + 2, row, 0.0))
        return cx, cy, cz

    def dist_to(cx, cy, cz):
        dx = X - cx
        dy = Y - cy
        dz = Z - cz
        return dx * dx + dy * dy + dz * dz

    cx0, cy0, cz0 = extract(jnp.int32(0))
    cent_ref[pl.ds(0, 1), :] = jnp.stack(
        [cx0, cy0, cz0, jnp.float32(0.0)])[None, :]
    mind0 = jnp.where(valid, dist_to(cx0, cy0, cz0), -jnp.inf)

    def body(i, mind):
        m = jnp.max(mind)
        nsel = jnp.min(jnp.where(mind == m, iota, jnp.int32(2**30)))
        cx, cy, cz = extract(nsel)
        cent_ref[pl.ds(i, 1), :] = jnp.stack(
            [cx, cy, cz, jnp.float32(0.0)])[None, :]
        d = dist_to(cx, cy, cz)
        return jnp.minimum(mind, d)

    jax.lax.fori_loop(1, N_BALLS, body, mind0, unroll=False)


def _fps(vertices):
    v = jnp.pad(vertices, ((0, N_PAD - N_PTS), (0, 0)))
    xw = v[:, 0].reshape(8, WCOLS)
    yw = v[:, 1].reshape(8, WCOLS)
    zw = v[:, 2].reshape(8, WCOLS)
    l = jnp.pad(v, ((0, 0), (0, 1))).reshape(N_PAD // 8, 32)
    cent = pl.pallas_call(
        _fps_body,
        out_shape=jax.ShapeDtypeStruct((N_BALLS, 4), jnp.float32),
    )(xw, yw, zw, l)
    return cent[:, :3]


def kernel(vertices, W1, b1, W2, b2, W3, b3):
    cent = _fps(vertices)
    sq = jnp.einsum('ij,ij->i', vertices, vertices)
    dist = jnp.sqrt(jnp.abs(sq[None, :] - 2.0 * (cent @ vertices.T) + sq[None, :]))
    neg_vals, nidx = jax.lax.top_k(-dist, K_NBR + 1)
    nd = -neg_vals
    limit = jnp.minimum(nd[:, K_NBR], RADIUS)
    mask = nd <= limit[:, None]
    neigh = vertices[nidx]
    rel = neigh - cent[:, None, :]
    feat = jnp.concatenate([neigh, rel], axis=-1)
    h = jax.nn.relu(feat @ W1 + b1) @ W2 + b2
    h = jnp.where(mask[..., None], h, -jnp.inf)
    agg = jnp.max(h, axis=1)
    return agg @ W3 + b3
